# baseline (speedup 1.0000x reference)
import functools

import jax
import jax.numpy as jnp
from jax import lax
from jax.experimental import pallas as pl
from jax.experimental.pallas import tpu as pltpu

_ROWS = 512


def _pair_body(nsteps, b, lower_ref, higher_ref, idx_ref, pLr, pHr, out,
               cl_s, ch_s, clT, chT, acc, sacc):
    i = pl.program_id(0)
    zero = jnp.float32(0.0)

    @pl.when(i == 0)
    def _init():
        def gather_one(r, _):
            row = idx_ref[r]
            cl_s[pl.ds(r, 1), :] = lower_ref[pl.ds(row, 1), :]
            ch_s[pl.ds(r, 1), :] = higher_ref[pl.ds(row, 1), :]
            return _

        lax.fori_loop(0, b, gather_one, 0, unroll=32)
        clT[...] = cl_s[...].T
        chT[...] = ch_s[...].T
        acc[...] = jnp.zeros_like(acc)
        cla = cl_s[...]
        cha = ch_s[...]
        plr = pLr[...]
        phr = pHr[...]
        exvec = (jnp.maximum(plr - cla, zero)
                 + jnp.maximum(cha - phr, zero)
                 + jnp.maximum(plr - cha, zero)
                 + jnp.maximum(cla - phr, zero))
        sacc[0] = jnp.sum(exvec)

    clb = cl_s[pl.ds(i * _ROWS, _ROWS), :]
    chb = ch_s[pl.ds(i * _ROWS, _ROWS), :]

    dchunk = 32
    ncb = _ROWS // 128
    iota_l = lax.broadcasted_iota(jnp.int32, (dchunk, 128), 1)
    for dc in range(0, clT.shape[0], dchunk):
        b_l = [clT[dc:dc + dchunk, cb * 128:(cb + 1) * 128] for cb in range(ncb)]
        b_h = [chT[dc:dc + dchunk, cb * 128:(cb + 1) * 128] for cb in range(ncb)]
        t = [acc[dc:dc + dchunk, cb * 128:(cb + 1) * 128] for cb in range(ncb)]
        for r in range(_ROWS):
            br, rloc = r // 128, r % 128
            a_l = clb[r, dc:dc + dchunk][:, None]
            a_h = chb[r, dc:dc + dchunk][:, None]
            ov = jnp.maximum(
                jnp.minimum(a_h, b_h[br]) - jnp.maximum(a_l, b_l[br]), zero)
            t[br] = t[br] + jnp.where(iota_l > rloc, ov, zero)
            for cb in range(br + 1, ncb):
                t[cb] = t[cb] + jnp.maximum(
                    jnp.minimum(a_h, b_h[cb]) - jnp.maximum(a_l, b_l[cb]),
                    zero)
        for cb in range(ncb):
            acc[dc:dc + dchunk, cb * 128:(cb + 1) * 128] = t[cb]

    @pl.when(i == nsteps - 1)
    def _fin():
        out[...] = (sacc[0] + 2.0 * jnp.sum(acc[...]))[None, None]


def _pair_call(lower, higher, idx, pL, pH, interpret=False):
    n, d = lower.shape
    b = idx.shape[0]
    nsteps = b // _ROWS
    return pl.pallas_call(
        functools.partial(_pair_body, nsteps, b),
        grid=(nsteps,),
        in_specs=[
            pl.BlockSpec((n, d), lambda i: (0, 0)),
            pl.BlockSpec((n, d), lambda i: (0, 0)),
            pl.BlockSpec(memory_space=pltpu.SMEM),
            pl.BlockSpec((1, d), lambda i: (0, 0)),
            pl.BlockSpec((1, d), lambda i: (0, 0)),
        ],
        out_specs=pl.BlockSpec((1, 1), lambda i: (0, 0)),
        out_shape=jax.ShapeDtypeStruct((1, 1), jnp.float32),
        scratch_shapes=[
            pltpu.VMEM((b, d), jnp.float32),
            pltpu.VMEM((b, d), jnp.float32),
            pltpu.VMEM((d, b), jnp.float32),
            pltpu.VMEM((d, b), jnp.float32),
            pltpu.VMEM((d, b), jnp.float32),
            pltpu.SMEM((1,), jnp.float32),
        ],
        interpret=interpret,
    )(lower, higher, idx, pL, pH)


def kernel(idIndexes, omegaEmb, epoch, childrenLowerEmbedding,
           childrenHigherEmbedding, parentL_, parentH_):
    d = childrenLowerEmbedding.shape[1]
    idx = idIndexes.astype(jnp.int32)
    out = _pair_call(childrenLowerEmbedding, childrenHigherEmbedding, idx,
                     parentL_.reshape(1, d), parentH_.reshape(1, d))
    return out[0, 0]
